# baseline (speedup 1.0000x reference)
import functools

import jax
import jax.numpy as jnp
from jax import lax
from jax.experimental import pallas as pl
from jax.experimental.pallas import tpu as pltpu
from jax.experimental.pallas import tpu_sc as plsc

KEEP_RATIO = 0.5


def _scores_body(x_ref, o_ref):
    o_ref[...] = jnp.sum(jnp.abs(x_ref[...]), axis=(2, 3))[None]


_TCB = 20
_CH = 2
_NBUF = 4


def _sc_scores_body(x_hbm, out_hbm, b0, b1, b2, b3, scores_v,
                    s0, s1, s2, s3):
    w = lax.axis_index("s") * 2 + lax.axis_index("c")
    b = x_hbm.shape[0]
    c = x_hbm.shape[1]
    lanes = 16
    nsc = b - _TCB
    bufs = (b0, b1, b2, b3)
    sems = (s0, s1, s2, s3)

    tail = jnp.where(lax.iota(jnp.int32, lanes) >= 8, 1.0, 0.0)

    def image_sum(buf, j):
        nrow = buf.shape[1]
        z = jnp.zeros((lanes,), jnp.float32)
        a0, a1, a2, a3 = z, z, z, z
        for r in range(nrow):
            a0 = a0 + jnp.abs(buf[j, r, 0:16])
            a1 = a1 + jnp.abs(buf[j, r, 16:32])
            a2 = a2 + jnp.abs(buf[j, r, 32:48])
            a3 = a3 + jnp.abs(buf[j, r, 40:56]) * tail
        return (a0 + a1) + (a2 + a3)

    @pl.when(w < 2 * nsc)
    def _work():
        bi = _TCB + w // 2
        half = w % 2
        chalf = c // 2
        cbase = half * chalf
        nchunk = chalf // _CH

        for t in range(_NBUF):
            pltpu.async_copy(x_hbm.at[bi, pl.ds(cbase + t * _CH, _CH)],
                             bufs[t], sems[t])

        def ring_step(k, _):
            for t in range(_NBUF):
                chunk = _NBUF * k + t
                cb = cbase + chunk * _CH
                pltpu.make_async_copy(x_hbm.at[bi, pl.ds(cb, _CH)],
                                      bufs[t], sems[t]).wait()
                accs = [image_sum(bufs[t], j) for j in range(_CH)]

                @pl.when(chunk + _NBUF < nchunk)
                def _pf():
                    pltpu.async_copy(
                        x_hbm.at[bi, pl.ds(cb + _NBUF * _CH, _CH)],
                        bufs[t], sems[t])

                for j in range(_CH):
                    scores_v[pl.ds((chunk * _CH + j) * 16, 16)] = accs[j]
            return 0

        lax.fori_loop(0, nchunk // _NBUF, ring_step, 0)
        pltpu.sync_copy(scores_v.at[pl.ds(0, chalf * 16)],
                        out_hbm.at[w // 2, pl.ds(cbase * 16, chalf * 16)])


def _sc_mask_body(bits_hbm, out_hbm, row_v, mask_v, tmp_v, lanes_sem):
    w = lax.axis_index("s") * 2 + lax.axis_index("c")
    c = bits_hbm.shape[1]
    lanes = 16
    nv = c // lanes
    k = max(1, min(c, int(round(c * KEEP_RATIO))))

    pltpu.sync_copy(bits_hbm.at[w], row_v)
    iota = lax.iota(jnp.int32, lanes)
    one = jnp.ones((lanes,), jnp.int32)
    zero = jnp.zeros((lanes,), jnp.int32)

    def lane_total(x):
        s = x
        for sh in (8, 4, 2, 1):
            tmp_v[pl.ds(0, lanes)] = s
            tmp_v[pl.ds(lanes, lanes)] = s
            s = s + tmp_v[pl.ds(sh, lanes)]
        return s

    def count_ge(thr):
        def body(i, cnt):
            v = row_v[pl.ds(i * lanes, lanes)]
            return cnt + jnp.where(v >= thr, one, zero)
        return lane_total(lax.fori_loop(0, nv, body, zero))

    def vsearch(_, carry):
        lo, hi = carry
        mid = lo + ((hi - lo + 1) >> 1)
        p = count_ge(mid) >= k
        return (jnp.where(p, mid, lo),
                jnp.where(p, hi, mid - jnp.ones((lanes,), jnp.int32)))

    lo = jnp.zeros((lanes,), jnp.int32)
    hi = jnp.full((lanes,), 0x7F800000, jnp.int32)
    t, _ = lax.fori_loop(0, 31, vsearch, (lo, hi))

    need_eq = k - count_ge(t + jnp.ones((lanes,), jnp.int32))

    def count_eq_le(m):
        def body(i, cnt):
            v = row_v[pl.ds(i * lanes, lanes)]
            col = iota + i * lanes
            return cnt + jnp.where((v == t) & (col <= m), one, zero)
        return lane_total(lax.fori_loop(0, nv, body, zero))

    def isearch(_, carry):
        lo2, hi2 = carry
        mid = (lo2 + hi2) >> 1
        p = count_eq_le(mid) >= need_eq
        return (jnp.where(p, lo2, mid + jnp.ones((lanes,), jnp.int32)),
                jnp.where(p, mid, hi2))

    lo2 = jnp.zeros((lanes,), jnp.int32)
    hi2 = jnp.full((lanes,), c - 1, jnp.int32)
    m, _ = lax.fori_loop(0, 10, isearch, (lo2, hi2))

    def write_mask(i, _):
        v = row_v[pl.ds(i * lanes, lanes)]
        col = iota + i * lanes
        keep = (v > t) | ((v == t) & (col <= m))
        mask_v[pl.ds(i * lanes, lanes)] = jnp.where(keep, 1.0, 0.0)
        return 0

    lax.fori_loop(0, nv, write_mask, 0)
    pltpu.sync_copy(mask_v, out_hbm.at[w])


def kernel(features, enabled):
    b, c, h, w = features.shape

    nsc = b - _TCB

    sc_reduce = functools.partial(
        pl.kernel,
        mesh=plsc.VectorSubcoreMesh(core_axis_name="c", subcore_axis_name="s"),
        out_type=jax.ShapeDtypeStruct((nsc, c * 16), jnp.float32),
        scratch_types=(
            [pltpu.VMEM((_CH, h, w), jnp.float32) for _ in range(_NBUF)]
            + [pltpu.VMEM((c // 2 * 16,), jnp.float32)]
            + [pltpu.SemaphoreType.DMA for _ in range(_NBUF)]
        ),
    )(_sc_scores_body)
    sc_partials = sc_reduce(features)
    sc_scores = jnp.sum(sc_partials.reshape(nsc, c, 16), axis=-1)

    bblk, cblk = 4, 128
    scores3 = pl.pallas_call(
        _scores_body,
        grid=(_TCB // bblk, c // cblk),
        in_specs=[pl.BlockSpec((bblk, cblk, h, w),
                               lambda i, j: (i, j, 0, 0))],
        out_specs=pl.BlockSpec((1, bblk, cblk), lambda i, j: (i, 0, j)),
        out_shape=jax.ShapeDtypeStruct((_TCB // bblk, bblk, c), jnp.float32),
        compiler_params=pltpu.CompilerParams(
            dimension_semantics=("parallel", "parallel")),
    )(features)
    scores = jnp.concatenate([scores3.reshape(_TCB, c), sc_scores], axis=0)
    bits = jax.lax.bitcast_convert_type(scores, jnp.int32)

    sc_mask = functools.partial(
        pl.kernel,
        mesh=plsc.VectorSubcoreMesh(core_axis_name="c", subcore_axis_name="s"),
        out_type=jax.ShapeDtypeStruct((b, c), jnp.float32),
        scratch_types=[
            pltpu.VMEM((c,), jnp.int32),
            pltpu.VMEM((c,), jnp.float32),
            pltpu.VMEM((32,), jnp.int32),
            pltpu.SemaphoreType.DMA,
        ],
    )(_sc_mask_body)
    mask = sc_mask(bits)

    gated = mask.reshape(b, c, 1, 1)
    return jnp.where(jnp.asarray(enabled) != 0, gated,
                     jnp.ones_like(gated))


# speedup vs baseline: 1.0171x; 1.0171x over previous
import functools

import jax
import jax.numpy as jnp
from jax import lax
from jax.experimental import pallas as pl
from jax.experimental.pallas import tpu as pltpu
from jax.experimental.pallas import tpu_sc as plsc

KEEP_RATIO = 0.5


def _scores_body(x_ref, o_ref):
    o_ref[...] = jnp.sum(jnp.abs(x_ref[...]), axis=(2, 3))[None]


def _sc_mask_body(bits_hbm, out_hbm, row_v, mask_v, tmp_v, lanes_sem):
    w = lax.axis_index("s") * 2 + lax.axis_index("c")
    c = bits_hbm.shape[1]
    lanes = 16
    nv = c // lanes
    k = max(1, min(c, int(round(c * KEEP_RATIO))))

    pltpu.sync_copy(bits_hbm.at[w], row_v)
    iota = lax.iota(jnp.int32, lanes)
    one = jnp.ones((lanes,), jnp.int32)
    zero = jnp.zeros((lanes,), jnp.int32)

    def lane_total(x):
        s = x
        for sh in (8, 4, 2, 1):
            tmp_v[pl.ds(0, lanes)] = s
            tmp_v[pl.ds(lanes, lanes)] = s
            s = s + tmp_v[pl.ds(sh, lanes)]
        return s

    def count_ge(thr):
        def body(i, cnt):
            v = row_v[pl.ds(i * lanes, lanes)]
            return cnt + jnp.where(v >= thr, one, zero)
        return lane_total(lax.fori_loop(0, nv, body, zero))

    def vsearch(_, carry):
        lo, hi = carry
        mid = lo + ((hi - lo + 1) >> 1)
        p = count_ge(mid) >= k
        return (jnp.where(p, mid, lo),
                jnp.where(p, hi, mid - jnp.ones((lanes,), jnp.int32)))

    lo = jnp.zeros((lanes,), jnp.int32)
    hi = jnp.full((lanes,), 0x7F800000, jnp.int32)
    t, _ = lax.fori_loop(0, 31, vsearch, (lo, hi))

    need_eq = k - count_ge(t + jnp.ones((lanes,), jnp.int32))

    def count_eq_le(m):
        def body(i, cnt):
            v = row_v[pl.ds(i * lanes, lanes)]
            col = iota + i * lanes
            return cnt + jnp.where((v == t) & (col <= m), one, zero)
        return lane_total(lax.fori_loop(0, nv, body, zero))

    def isearch(_, carry):
        lo2, hi2 = carry
        mid = (lo2 + hi2) >> 1
        p = count_eq_le(mid) >= need_eq
        return (jnp.where(p, lo2, mid + jnp.ones((lanes,), jnp.int32)),
                jnp.where(p, mid, hi2))

    lo2 = jnp.zeros((lanes,), jnp.int32)
    hi2 = jnp.full((lanes,), c - 1, jnp.int32)
    m, _ = lax.fori_loop(0, 10, isearch, (lo2, hi2))

    def write_mask(i, _):
        v = row_v[pl.ds(i * lanes, lanes)]
        col = iota + i * lanes
        keep = (v > t) | ((v == t) & (col <= m))
        mask_v[pl.ds(i * lanes, lanes)] = jnp.where(keep, 1.0, 0.0)
        return 0

    lax.fori_loop(0, nv, write_mask, 0)
    pltpu.sync_copy(mask_v, out_hbm.at[w])


def kernel(features, enabled):
    b, c, h, w = features.shape

    bblk, cblk = 4, 128
    scores3 = pl.pallas_call(
        _scores_body,
        grid=(b // bblk, c // cblk),
        in_specs=[pl.BlockSpec((bblk, cblk, h, w),
                               lambda i, j: (i, j, 0, 0))],
        out_specs=pl.BlockSpec((1, bblk, cblk), lambda i, j: (i, 0, j)),
        out_shape=jax.ShapeDtypeStruct((b // bblk, bblk, c), jnp.float32),
        compiler_params=pltpu.CompilerParams(
            dimension_semantics=("parallel", "parallel")),
    )(features)
    bits = jax.lax.bitcast_convert_type(scores3.reshape(b, c), jnp.int32)

    sc_mask = functools.partial(
        pl.kernel,
        mesh=plsc.VectorSubcoreMesh(core_axis_name="c", subcore_axis_name="s"),
        out_type=jax.ShapeDtypeStruct((b, c), jnp.float32),
        scratch_types=[
            pltpu.VMEM((c,), jnp.int32),
            pltpu.VMEM((c,), jnp.float32),
            pltpu.VMEM((32,), jnp.int32),
            pltpu.SemaphoreType.DMA,
        ],
    )(_sc_mask_body)
    mask = sc_mask(bits)

    gated = mask.reshape(b, c, 1, 1)
    return jnp.where(jnp.asarray(enabled) != 0, gated,
                     jnp.ones_like(gated))
